# baseline (speedup 1.0000x reference)
import jax
import jax.numpy as jnp
from jax.experimental import pallas as pl
from jax.experimental.pallas import tpu as pltpu

_R, _C = 2048, 4096
_CHUNK = 2048
_NCHUNK = _R // _CHUNK


def _copy_body(in_ref, out_ref, *scratch):
    bufs = scratch[:_NCHUNK]
    isems = scratch[_NCHUNK:2 * _NCHUNK]
    osems = scratch[2 * _NCHUNK:]
    in2d = in_ref.reshape(_R, _C)

    def rows(c):
        return pl.ds(c * _CHUNK, _CHUNK)

    ins = []
    for c in range(_NCHUNK):
        cp = pltpu.make_async_copy(in2d.at[rows(c)], bufs[c], isems[c])
        cp.start()
        ins.append(cp)
    outs = []
    for c in range(_NCHUNK):
        ins[c].wait()
        cp = pltpu.make_async_copy(bufs[c], out_ref.at[rows(c)], osems[c])
        cp.start()
        outs.append(cp)
    for cp in outs:
        cp.wait()


def kernel(free_values):
    x = free_values.reshape(_R * _C // 128, 128)
    return pl.pallas_call(
        _copy_body,
        in_specs=[pl.BlockSpec(memory_space=pl.ANY)],
        out_specs=pl.BlockSpec(memory_space=pl.ANY),
        out_shape=jax.ShapeDtypeStruct((_R, _C), jnp.float32),
        scratch_shapes=(
            [pltpu.VMEM((_CHUNK, _C), jnp.float32) for _ in range(_NCHUNK)]
            + [pltpu.SemaphoreType.DMA for _ in range(2 * _NCHUNK)]
        ),
    )(x)


# speedup vs baseline: 1.0503x; 1.0503x over previous
import jax
import jax.numpy as jnp
from jax.experimental import pallas as pl
from jax.experimental.pallas import tpu as pltpu

_R, _C = 2048, 4096
_INCHUNK = 1024
_NIN = _R // _INCHUNK
_OUTCHUNK = 256
_OUT_PER_IN = _INCHUNK // _OUTCHUNK


def _copy_body(in_ref, out_ref, *scratch):
    bufs = scratch[:_NIN]
    isems = scratch[_NIN:2 * _NIN]
    osems = scratch[2 * _NIN:]
    in2d = in_ref.reshape(_R, _C)

    ins = []
    for c in range(_NIN):
        cp = pltpu.make_async_copy(
            in2d.at[pl.ds(c * _INCHUNK, _INCHUNK)], bufs[c], isems[c])
        cp.start()
        ins.append(cp)
    outs = []
    for c in range(_NIN):
        ins[c].wait()
        for j in range(_OUT_PER_IN):
            k = c * _OUT_PER_IN + j
            cp = pltpu.make_async_copy(
                bufs[c].at[pl.ds(j * _OUTCHUNK, _OUTCHUNK)],
                out_ref.at[pl.ds(k * _OUTCHUNK, _OUTCHUNK)],
                osems[k],
            )
            cp.start()
            outs.append(cp)
    for cp in outs:
        cp.wait()


def kernel(free_values):
    x = free_values.reshape(_R * _C // 128, 128)
    return pl.pallas_call(
        _copy_body,
        in_specs=[pl.BlockSpec(memory_space=pl.ANY)],
        out_specs=pl.BlockSpec(memory_space=pl.ANY),
        out_shape=jax.ShapeDtypeStruct((_R, _C), jnp.float32),
        scratch_shapes=(
            [pltpu.VMEM((_INCHUNK, _C), jnp.float32) for _ in range(_NIN)]
            + [pltpu.SemaphoreType.DMA for _ in range(_NIN)]
            + [pltpu.SemaphoreType.DMA for _ in range(_NIN * _OUT_PER_IN)]
        ),
    )(x)


# speedup vs baseline: 1.0564x; 1.0058x over previous
import jax
import jax.numpy as jnp
from jax.experimental import pallas as pl
from jax.experimental.pallas import tpu as pltpu

_R, _C = 2048, 4096
_CHUNK = 1024
_NCHUNK = _R // _CHUNK


def _copy_body(in_ref, out_ref, *scratch):
    bufs = scratch[:_NCHUNK]
    isems = scratch[_NCHUNK:2 * _NCHUNK]
    osems = scratch[2 * _NCHUNK:]
    in2d = in_ref.reshape(_R, _C)

    def rows(c):
        return pl.ds(c * _CHUNK, _CHUNK)

    ins = []
    for c in range(_NCHUNK):
        cp = pltpu.make_async_copy(in2d.at[rows(c)], bufs[c], isems[c])
        cp.start()
        ins.append(cp)
    outs = []
    for c in range(_NCHUNK):
        ins[c].wait()
        cp = pltpu.make_async_copy(bufs[c], out_ref.at[rows(c)], osems[c])
        cp.start()
        outs.append(cp)
    for cp in outs:
        cp.wait()


def kernel(free_values):
    x = free_values.reshape(_R * _C // 128, 128)
    return pl.pallas_call(
        _copy_body,
        in_specs=[pl.BlockSpec(memory_space=pl.ANY)],
        out_specs=pl.BlockSpec(memory_space=pl.ANY),
        out_shape=jax.ShapeDtypeStruct((_R, _C), jnp.float32),
        scratch_shapes=(
            [pltpu.VMEM((_CHUNK, _C), jnp.float32) for _ in range(_NCHUNK)]
            + [pltpu.SemaphoreType.DMA for _ in range(2 * _NCHUNK)]
        ),
    )(x)
